# baseline (speedup 1.0000x reference)
import functools

import jax
import jax.numpy as jnp
from jax import lax
from jax.experimental import pallas as pl
from jax.experimental.pallas import tpu as pltpu
from jax.experimental.pallas import tpu_sc as plsc

N = 10000
E = 320000
D = 128
NT = 16
CH = 128
NCH = 160
GRP = 16
NGRP = NCH // GRP
EPT = NCH * CH
EPAD = NT * EPT
NPAD = 10016
CB = 632
CBL = N - 15 * CB
CBLZ = NPAD - 15 * CB


@functools.cache
def _get_deg_kernel():
    mesh = plsc.VectorSubcoreMesh(core_axis_name="c", subcore_axis_name="s")
    return functools.partial(
        pl.kernel,
        mesh=mesh,
        out_type=jax.ShapeDtypeStruct((2, N, D), jnp.float32),
        scratch_types=[
            pltpu.VMEM((GRP, CH), jnp.int32),
            pltpu.VMEM((CH, D), jnp.float32),
            pltpu.VMEM_SHARED((NPAD, D), jnp.float32),
        ],
    )(_deg_body)


def _deg_body(dst_hbm, ones_hbm, out_hbm, dst_v, rows_v, acc_sh):
    c = lax.axis_index("c")
    s = lax.axis_index("s")
    pltpu.sync_copy(ones_hbm.at[pl.ds(0, CH)], rows_v)

    @pl.when(s < 15)
    def _():
        pltpu.sync_copy(ones_hbm.at[pl.ds(s * CB, CB)], acc_sh.at[pl.ds(s * CB, CB)])

    @pl.when(s == 15)
    def _():
        pltpu.sync_copy(ones_hbm.at[pl.ds(15 * CB, CBL)], acc_sh.at[pl.ds(15 * CB, CBL)])

    plsc.subcore_barrier()

    def grp_body(g, carry):
        pltpu.sync_copy(dst_hbm.at[c, s, pl.ds(g * GRP, GRP)], dst_v)

        def body(k, carry2):
            pltpu.sync_copy(rows_v, acc_sh.at[dst_v.at[k]], add=True)
            return carry2

        lax.fori_loop(0, GRP, body, 0)
        return carry

    lax.fori_loop(0, NGRP, grp_body, 0)
    plsc.subcore_barrier()

    @pl.when(s < 15)
    def _():
        pltpu.sync_copy(acc_sh.at[pl.ds(s * CB, CB)], out_hbm.at[c, pl.ds(s * CB, CB)])

    @pl.when(s == 15)
    def _():
        pltpu.sync_copy(acc_sh.at[pl.ds(15 * CB, CBL)], out_hbm.at[c, pl.ds(15 * CB, CBL)])


@functools.cache
def _get_agg_kernel():
    mesh = plsc.VectorSubcoreMesh(core_axis_name="c", subcore_axis_name="s")
    return functools.partial(
        pl.kernel,
        mesh=mesh,
        out_type=jax.ShapeDtypeStruct((2, N, D), jnp.float32),
        scratch_types=[
            pltpu.VMEM((GRP, CH), jnp.int32),
            pltpu.VMEM((GRP, CH), jnp.int32),
            pltpu.VMEM((CH, D), jnp.float32),
            pltpu.VMEM((CH, D), jnp.float32),
            pltpu.VMEM_SHARED((NPAD, D), jnp.float32),
            pltpu.SemaphoreType.DMA,
            pltpu.SemaphoreType.DMA,
            pltpu.SemaphoreType.DMA,
            pltpu.SemaphoreType.DMA,
        ],
    )(_agg_body)


def _agg_body(hp_hbm, src_hbm, dst_hbm, out_hbm, src_v, dst_v, rows_a, rows_b,
              acc_sh, sem_ga, sem_gb, sem_sa, sem_sb):
    c = lax.axis_index("c")
    s = lax.axis_index("s")
    hp_c = hp_hbm.at[c]

    @pl.when(s < 15)
    def _():
        pltpu.sync_copy(hp_c.at[pl.ds(s * CB, CB)], acc_sh.at[pl.ds(s * CB, CB)])

    @pl.when(s == 15)
    def _():
        pltpu.sync_copy(hp_c.at[pl.ds(15 * CB, CBL)], acc_sh.at[pl.ds(15 * CB, CBL)])

    plsc.subcore_barrier()

    def grp_body(g, carry):
        pltpu.sync_copy(src_hbm.at[c, s, pl.ds(g * GRP, GRP)], src_v)
        pltpu.sync_copy(dst_hbm.at[c, s, pl.ds(g * GRP, GRP)], dst_v)
        pltpu.async_copy(hp_c.at[src_v.at[0]], rows_a, sem_ga)
        pltpu.async_copy(hp_c.at[src_v.at[1]], rows_b, sem_gb)

        def pair(kk, carry2):
            k0 = kk * 2
            pltpu.make_async_copy(hp_c.at[src_v.at[k0]], rows_a, sem_ga).wait()
            pltpu.async_copy(rows_a, acc_sh.at[dst_v.at[k0]], sem_sa, add=True)
            pltpu.make_async_copy(hp_c.at[src_v.at[k0 + 1]], rows_b, sem_gb).wait()
            pltpu.async_copy(rows_b, acc_sh.at[dst_v.at[k0 + 1]], sem_sb, add=True)
            pltpu.make_async_copy(rows_a, acc_sh.at[dst_v.at[k0]], sem_sa).wait()

            @pl.when(kk < GRP // 2 - 1)
            def _():
                pltpu.async_copy(hp_c.at[src_v.at[k0 + 2]], rows_a, sem_ga)

            pltpu.make_async_copy(rows_b, acc_sh.at[dst_v.at[k0 + 1]], sem_sb).wait()

            @pl.when(kk < GRP // 2 - 1)
            def _():
                pltpu.async_copy(hp_c.at[src_v.at[k0 + 3]], rows_b, sem_gb)

            return carry2

        lax.fori_loop(0, GRP // 2, pair, 0)
        return carry

    lax.fori_loop(0, NGRP, grp_body, 0)
    plsc.subcore_barrier()

    @pl.when(s < 15)
    def _():
        pltpu.sync_copy(acc_sh.at[pl.ds(s * CB, CB)], out_hbm.at[c, pl.ds(s * CB, CB)])

    @pl.when(s == 15)
    def _():
        pltpu.sync_copy(acc_sh.at[pl.ds(15 * CB, CBL)], out_hbm.at[c, pl.ds(15 * CB, CBL)])


_BM = 1000
_NB = N // _BM


def _tc1_body(x_ref, w_ref, deg_ref, hp_ref):
    dv = lax.rsqrt(deg_ref[0])
    h = jnp.dot(x_ref[0], w_ref[...], preferred_element_type=jnp.float32)
    hp_ref[0] = h * dv


def _tc1(x, w1, deg):
    return pl.pallas_call(
        _tc1_body,
        grid=(2, _NB),
        in_specs=[
            pl.BlockSpec((1, _BM, D), lambda g, i: (g, i, 0)),
            pl.BlockSpec((D, D), lambda g, i: (0, 0)),
            pl.BlockSpec((1, _BM, D), lambda g, i: (g, i, 0)),
        ],
        out_specs=pl.BlockSpec((1, _BM, D), lambda g, i: (g, i, 0)),
        out_shape=jax.ShapeDtypeStruct((2, N, D), jnp.float32),
    )(x, w1, deg)


def _tc2_body(agg_ref, deg_ref, b_ref, w_ref, out_ref):
    dv = lax.rsqrt(deg_ref[0])
    y = jnp.maximum(agg_ref[0] * dv + b_ref[...], 0.0)
    out_ref[0] = jnp.dot(y, w_ref[...], preferred_element_type=jnp.float32) * dv


def _tc2(agg, deg, b1, w2):
    return pl.pallas_call(
        _tc2_body,
        grid=(2, _NB),
        in_specs=[
            pl.BlockSpec((1, _BM, D), lambda g, i: (g, i, 0)),
            pl.BlockSpec((1, _BM, D), lambda g, i: (g, i, 0)),
            pl.BlockSpec((1, D), lambda g, i: (0, 0)),
            pl.BlockSpec((D, D), lambda g, i: (0, 0)),
        ],
        out_specs=pl.BlockSpec((1, _BM, D), lambda g, i: (g, i, 0)),
        out_shape=jax.ShapeDtypeStruct((2, N, D), jnp.float32),
    )(agg, deg, b1, w2)


def _tc3_body(agg_ref, deg_ref, b_ref, out_ref, sum_ref, sq_ref):
    p = pl.program_id(1)
    i = pl.program_id(2)
    dv = lax.rsqrt(deg_ref[0])
    h = agg_ref[0] * dv + b_ref[...]

    @pl.when((p == 0) & (i == 0))
    def _():
        sum_ref[...] = jnp.zeros_like(sum_ref)
        sq_ref[...] = jnp.zeros_like(sq_ref)

    @pl.when(p == 0)
    def _():
        sum_ref[...] += jnp.sum(h, axis=0, keepdims=True)
        sq_ref[...] += jnp.sum(h * h, axis=0, keepdims=True)
        out_ref[0] = h

    @pl.when(p == 1)
    def _():
        mean = sum_ref[...] / N
        var = (sq_ref[...] - N * mean * mean) / (N - 1)
        out_ref[0] = (h - mean) * lax.rsqrt(var)


def _tc3(agg, deg, b2):
    return pl.pallas_call(
        _tc3_body,
        grid=(2, 2, _NB),
        in_specs=[
            pl.BlockSpec((1, _BM, D), lambda g, p, i: (g, i, 0)),
            pl.BlockSpec((1, _BM, D), lambda g, p, i: (g, i, 0)),
            pl.BlockSpec((1, D), lambda g, p, i: (0, 0)),
        ],
        out_specs=pl.BlockSpec((1, _BM, D), lambda g, p, i: (g, i, 0)),
        out_shape=jax.ShapeDtypeStruct((2, N, D), jnp.float32),
        scratch_shapes=[
            pltpu.VMEM((1, D), jnp.float32),
            pltpu.VMEM((1, D), jnp.float32),
        ],
    )(agg, deg, b2)


def kernel(x1, x2, edge_index1, edge_index2, W1, b1, W2, b2):
    src = jnp.stack([edge_index1[0], edge_index2[0]])
    dst = jnp.stack([edge_index1[1], edge_index2[1]])
    extra = EPAD - E
    pad_src = (jnp.arange(extra, dtype=jnp.int32) * 97) % N
    pad_dst = N + (jnp.arange(extra, dtype=jnp.int32) % (NPAD - N))
    srcp = jnp.concatenate(
        [src, jnp.broadcast_to(pad_src, (2, extra))], axis=1
    ).reshape(2, NT, NCH, CH)
    dstp = jnp.concatenate(
        [dst, jnp.broadcast_to(pad_dst, (2, extra))], axis=1
    ).reshape(2, NT, NCH, CH)

    ones = jnp.ones((N, D), jnp.float32)
    x = jnp.stack([x1, x2])

    deg = _get_deg_kernel()(dstp, ones)
    hp1 = _tc1(x, W1, deg)
    agg = _get_agg_kernel()
    agg1 = agg(hp1, srcp, dstp)
    hp2 = _tc2(agg1, deg, b1.reshape(1, D), W2)
    agg2 = agg(hp2, srcp, dstp)
    z = _tc3(agg2, deg, b2.reshape(1, D))
    return z[0], z[1]


# speedup vs baseline: 1.2281x; 1.2281x over previous
import functools

import jax
import jax.numpy as jnp
from jax import lax
from jax.experimental import pallas as pl
from jax.experimental.pallas import tpu as pltpu
from jax.experimental.pallas import tpu_sc as plsc

N = 10000
E = 320000
D = 128
NT = 16
CH = 128
NCH = 160
GRP = 16
NGRP = NCH // GRP
EPT = NCH * CH
EPAD = NT * EPT
NPAD = 10016
CB = 632
CBL = N - 15 * CB
CBLZ = NPAD - 15 * CB


@functools.cache
def _get_deg_kernel():
    mesh = plsc.VectorSubcoreMesh(core_axis_name="c", subcore_axis_name="s")
    return functools.partial(
        pl.kernel,
        mesh=mesh,
        out_type=jax.ShapeDtypeStruct((2, N, D), jnp.float32),
        scratch_types=[
            pltpu.VMEM((GRP, CH), jnp.int32),
            pltpu.VMEM((CH, D), jnp.float32),
            pltpu.VMEM_SHARED((NPAD, D), jnp.float32),
        ],
    )(_deg_body)


def _deg_body(dst_hbm, ones_hbm, out_hbm, dst_v, rows_v, acc_sh):
    c = lax.axis_index("c")
    s = lax.axis_index("s")
    pltpu.sync_copy(ones_hbm.at[pl.ds(0, CH)], rows_v)

    @pl.when(s < 15)
    def _():
        pltpu.sync_copy(ones_hbm.at[pl.ds(s * CB, CB)], acc_sh.at[pl.ds(s * CB, CB)])

    @pl.when(s == 15)
    def _():
        pltpu.sync_copy(ones_hbm.at[pl.ds(15 * CB, CBL)], acc_sh.at[pl.ds(15 * CB, CBL)])

    plsc.subcore_barrier()

    def grp_body(g, carry):
        pltpu.sync_copy(dst_hbm.at[c, s, pl.ds(g * GRP, GRP)], dst_v)

        def body(k, carry2):
            pltpu.sync_copy(rows_v, acc_sh.at[dst_v.at[k]], add=True)
            return carry2

        lax.fori_loop(0, GRP, body, 0)
        return carry

    lax.fori_loop(0, NGRP, grp_body, 0)
    plsc.subcore_barrier()

    @pl.when(s < 15)
    def _():
        pltpu.sync_copy(acc_sh.at[pl.ds(s * CB, CB)], out_hbm.at[c, pl.ds(s * CB, CB)])

    @pl.when(s == 15)
    def _():
        pltpu.sync_copy(acc_sh.at[pl.ds(15 * CB, CBL)], out_hbm.at[c, pl.ds(15 * CB, CBL)])


@functools.cache
def _get_agg_kernel():
    mesh = plsc.VectorSubcoreMesh(core_axis_name="c", subcore_axis_name="s")
    return functools.partial(
        pl.kernel,
        mesh=mesh,
        out_type=jax.ShapeDtypeStruct((2, N, D), jnp.float32),
        scratch_types=[
            pltpu.VMEM((GRP, CH), jnp.int32),
            pltpu.VMEM((GRP, CH), jnp.int32),
            pltpu.VMEM((GRP, CH), jnp.int32),
            pltpu.VMEM((GRP, CH), jnp.int32),
            pltpu.VMEM((CH, D), jnp.float32),
            pltpu.VMEM((CH, D), jnp.float32),
            pltpu.VMEM_SHARED((NPAD, D), jnp.float32),
            pltpu.SemaphoreType.DMA,
            pltpu.SemaphoreType.DMA,
            pltpu.SemaphoreType.DMA,
            pltpu.SemaphoreType.DMA,
        ],
    )(_agg_body)


def _agg_body(hp_hbm, src_hbm, dst_hbm, out_hbm, src_a, dst_a, src_b, dst_b,
              rows_a, rows_b, acc_sh, sem_ga, sem_gb, sem_ia, sem_ib):
    c = lax.axis_index("c")
    s = lax.axis_index("s")
    hp_c = hp_hbm.at[c]

    @pl.when(s < 15)
    def _():
        pltpu.sync_copy(hp_c.at[pl.ds(s * CB, CB)], acc_sh.at[pl.ds(s * CB, CB)])

    @pl.when(s == 15)
    def _():
        pltpu.sync_copy(hp_c.at[pl.ds(15 * CB, CBL)], acc_sh.at[pl.ds(15 * CB, CBL)])

    plsc.subcore_barrier()

    def load_idx(g, sv, dv, sem):
        pltpu.async_copy(src_hbm.at[c, s, pl.ds(g * GRP, GRP)], sv, sem)
        pltpu.async_copy(dst_hbm.at[c, s, pl.ds(g * GRP, GRP)], dv, sem)

    def wait_idx(g, sv, dv, sem):
        pltpu.make_async_copy(src_hbm.at[c, s, pl.ds(g * GRP, GRP)], sv, sem).wait()
        pltpu.make_async_copy(dst_hbm.at[c, s, pl.ds(g * GRP, GRP)], dv, sem).wait()

    def run_group(src_v, dst_v):
        pltpu.async_copy(hp_c.at[src_v.at[0]], rows_a, sem_ga)

        def pair(kk, carry2):
            k0 = kk * 2
            pltpu.async_copy(hp_c.at[src_v.at[k0 + 1]], rows_b, sem_gb)
            pltpu.make_async_copy(hp_c.at[src_v.at[k0]], rows_a, sem_ga).wait()
            pltpu.sync_copy(rows_a, acc_sh.at[dst_v.at[k0]], add=True)

            @pl.when(kk < GRP // 2 - 1)
            def _():
                pltpu.async_copy(hp_c.at[src_v.at[k0 + 2]], rows_a, sem_ga)

            pltpu.make_async_copy(hp_c.at[src_v.at[k0 + 1]], rows_b, sem_gb).wait()
            pltpu.sync_copy(rows_b, acc_sh.at[dst_v.at[k0 + 1]], add=True)
            return carry2

        lax.fori_loop(0, GRP // 2, pair, 0)

    load_idx(0, src_a, dst_a, sem_ia)

    def grp_pair(gg, carry):
        g0 = gg * 2
        wait_idx(g0, src_a, dst_a, sem_ia)
        pltpu.async_copy(src_hbm.at[c, s, pl.ds((g0 + 1) * GRP, GRP)], src_b, sem_ib)
        pltpu.async_copy(dst_hbm.at[c, s, pl.ds((g0 + 1) * GRP, GRP)], dst_b, sem_ib)
        run_group(src_a, dst_a)
        wait_idx(g0 + 1, src_b, dst_b, sem_ib)

        @pl.when(gg < NGRP // 2 - 1)
        def _():
            load_idx(g0 + 2, src_a, dst_a, sem_ia)

        run_group(src_b, dst_b)
        return carry

    lax.fori_loop(0, NGRP // 2, grp_pair, 0)
    plsc.subcore_barrier()

    @pl.when(s < 15)
    def _():
        pltpu.sync_copy(acc_sh.at[pl.ds(s * CB, CB)], out_hbm.at[c, pl.ds(s * CB, CB)])

    @pl.when(s == 15)
    def _():
        pltpu.sync_copy(acc_sh.at[pl.ds(15 * CB, CBL)], out_hbm.at[c, pl.ds(15 * CB, CBL)])


_BM = 1000
_NB = N // _BM


def _tc1_body(x_ref, w_ref, deg_ref, hp_ref):
    dv = lax.rsqrt(deg_ref[0])
    h = jnp.dot(x_ref[0], w_ref[...], preferred_element_type=jnp.float32)
    hp_ref[0] = h * dv


def _tc1(x, w1, deg):
    return pl.pallas_call(
        _tc1_body,
        grid=(2, _NB),
        in_specs=[
            pl.BlockSpec((1, _BM, D), lambda g, i: (g, i, 0)),
            pl.BlockSpec((D, D), lambda g, i: (0, 0)),
            pl.BlockSpec((1, _BM, D), lambda g, i: (g, i, 0)),
        ],
        out_specs=pl.BlockSpec((1, _BM, D), lambda g, i: (g, i, 0)),
        out_shape=jax.ShapeDtypeStruct((2, N, D), jnp.float32),
    )(x, w1, deg)


def _tc2_body(agg_ref, deg_ref, b_ref, w_ref, out_ref):
    dv = lax.rsqrt(deg_ref[0])
    y = jnp.maximum(agg_ref[0] * dv + b_ref[...], 0.0)
    out_ref[0] = jnp.dot(y, w_ref[...], preferred_element_type=jnp.float32) * dv


def _tc2(agg, deg, b1, w2):
    return pl.pallas_call(
        _tc2_body,
        grid=(2, _NB),
        in_specs=[
            pl.BlockSpec((1, _BM, D), lambda g, i: (g, i, 0)),
            pl.BlockSpec((1, _BM, D), lambda g, i: (g, i, 0)),
            pl.BlockSpec((1, D), lambda g, i: (0, 0)),
            pl.BlockSpec((D, D), lambda g, i: (0, 0)),
        ],
        out_specs=pl.BlockSpec((1, _BM, D), lambda g, i: (g, i, 0)),
        out_shape=jax.ShapeDtypeStruct((2, N, D), jnp.float32),
    )(agg, deg, b1, w2)


def _tc3_body(agg_ref, deg_ref, b_ref, out_ref, sum_ref, sq_ref):
    p = pl.program_id(1)
    i = pl.program_id(2)
    dv = lax.rsqrt(deg_ref[0])
    h = agg_ref[0] * dv + b_ref[...]

    @pl.when((p == 0) & (i == 0))
    def _():
        sum_ref[...] = jnp.zeros_like(sum_ref)
        sq_ref[...] = jnp.zeros_like(sq_ref)

    @pl.when(p == 0)
    def _():
        sum_ref[...] += jnp.sum(h, axis=0, keepdims=True)
        sq_ref[...] += jnp.sum(h * h, axis=0, keepdims=True)
        out_ref[0] = h

    @pl.when(p == 1)
    def _():
        mean = sum_ref[...] / N
        var = (sq_ref[...] - N * mean * mean) / (N - 1)
        out_ref[0] = (h - mean) * lax.rsqrt(var)


def _tc3(agg, deg, b2):
    return pl.pallas_call(
        _tc3_body,
        grid=(2, 2, _NB),
        in_specs=[
            pl.BlockSpec((1, _BM, D), lambda g, p, i: (g, i, 0)),
            pl.BlockSpec((1, _BM, D), lambda g, p, i: (g, i, 0)),
            pl.BlockSpec((1, D), lambda g, p, i: (0, 0)),
        ],
        out_specs=pl.BlockSpec((1, _BM, D), lambda g, p, i: (g, i, 0)),
        out_shape=jax.ShapeDtypeStruct((2, N, D), jnp.float32),
        scratch_shapes=[
            pltpu.VMEM((1, D), jnp.float32),
            pltpu.VMEM((1, D), jnp.float32),
        ],
    )(agg, deg, b2)


def kernel(x1, x2, edge_index1, edge_index2, W1, b1, W2, b2):
    src = jnp.stack([edge_index1[0], edge_index2[0]])
    dst = jnp.stack([edge_index1[1], edge_index2[1]])
    extra = EPAD - E
    pad_src = (jnp.arange(extra, dtype=jnp.int32) * 97) % N
    pad_dst = N + (jnp.arange(extra, dtype=jnp.int32) % (NPAD - N))
    srcp = jnp.concatenate(
        [src, jnp.broadcast_to(pad_src, (2, extra))], axis=1
    ).reshape(2, NT, NCH, CH)
    dstp = jnp.concatenate(
        [dst, jnp.broadcast_to(pad_dst, (2, extra))], axis=1
    ).reshape(2, NT, NCH, CH)

    ones = jnp.ones((N, D), jnp.float32)
    x = jnp.stack([x1, x2])

    deg = _get_deg_kernel()(dstp, ones)
    hp1 = _tc1(x, W1, deg)
    agg = _get_agg_kernel()
    agg1 = agg(hp1, srcp, dstp)
    hp2 = _tc2(agg1, deg, b1.reshape(1, D), W2)
    agg2 = agg(hp2, srcp, dstp)
    z = _tc3(agg2, deg, b2.reshape(1, D))
    return z[0], z[1]


# speedup vs baseline: 1.2783x; 1.0408x over previous
import functools

import jax
import jax.numpy as jnp
from jax import lax
from jax.experimental import pallas as pl
from jax.experimental.pallas import tpu as pltpu
from jax.experimental.pallas import tpu_sc as plsc

N = 10000
E = 320000
D = 128
NT = 16
CH = 128
NCH = 160
GRP = 16
NGRP = NCH // GRP
EPT = NCH * CH
EPAD = NT * EPT
NPAD = 10016
CB = 632
CBL = N - 15 * CB
CBLZ = NPAD - 15 * CB


@functools.cache
def _get_deg_kernel():
    mesh = plsc.VectorSubcoreMesh(core_axis_name="c", subcore_axis_name="s")
    return functools.partial(
        pl.kernel,
        mesh=mesh,
        out_type=jax.ShapeDtypeStruct((2, N, D), jnp.float32),
        scratch_types=[
            pltpu.VMEM((GRP, CH), jnp.int32),
            pltpu.VMEM((GRP, CH), jnp.int32),
            pltpu.VMEM((CH, D), jnp.float32),
            pltpu.VMEM_SHARED((NPAD, D), jnp.float32),
            pltpu.SemaphoreType.DMA,
            pltpu.SemaphoreType.DMA,
        ],
    )(_deg_body)


def _deg_body(dst_hbm, ones_hbm, out_hbm, dst_a, dst_b, rows_v, acc_sh, sem_ia, sem_ib):
    c = lax.axis_index("c")
    s = lax.axis_index("s")
    pltpu.sync_copy(ones_hbm.at[pl.ds(0, CH)], rows_v)

    @pl.when(s < 15)
    def _():
        pltpu.sync_copy(ones_hbm.at[pl.ds(s * CB, CB)], acc_sh.at[pl.ds(s * CB, CB)])

    @pl.when(s == 15)
    def _():
        pltpu.sync_copy(ones_hbm.at[pl.ds(15 * CB, CBL)], acc_sh.at[pl.ds(15 * CB, CBL)])

    plsc.subcore_barrier()

    def run_group(dst_v):
        def body(k, carry2):
            pltpu.sync_copy(rows_v, acc_sh.at[dst_v.at[k]], add=True)
            return carry2

        lax.fori_loop(0, GRP, body, 0)

    pltpu.async_copy(dst_hbm.at[c, s, pl.ds(0, GRP)], dst_a, sem_ia)

    def grp_pair(gg, carry):
        g0 = gg * 2
        pltpu.make_async_copy(dst_hbm.at[c, s, pl.ds(g0 * GRP, GRP)], dst_a, sem_ia).wait()
        pltpu.async_copy(dst_hbm.at[c, s, pl.ds((g0 + 1) * GRP, GRP)], dst_b, sem_ib)
        run_group(dst_a)
        pltpu.make_async_copy(dst_hbm.at[c, s, pl.ds((g0 + 1) * GRP, GRP)], dst_b, sem_ib).wait()

        @pl.when(gg < NGRP // 2 - 1)
        def _():
            pltpu.async_copy(dst_hbm.at[c, s, pl.ds((g0 + 2) * GRP, GRP)], dst_a, sem_ia)

        run_group(dst_b)
        return carry

    lax.fori_loop(0, NGRP // 2, grp_pair, 0)
    plsc.subcore_barrier()

    @pl.when(s < 15)
    def _():
        pltpu.sync_copy(acc_sh.at[pl.ds(s * CB, CB)], out_hbm.at[c, pl.ds(s * CB, CB)])

    @pl.when(s == 15)
    def _():
        pltpu.sync_copy(acc_sh.at[pl.ds(15 * CB, CBL)], out_hbm.at[c, pl.ds(15 * CB, CBL)])


@functools.cache
def _get_agg_kernel():
    mesh = plsc.VectorSubcoreMesh(core_axis_name="c", subcore_axis_name="s")
    return functools.partial(
        pl.kernel,
        mesh=mesh,
        out_type=jax.ShapeDtypeStruct((2, N, D), jnp.float32),
        scratch_types=[
            pltpu.VMEM((GRP, CH), jnp.int32),
            pltpu.VMEM((GRP, CH), jnp.int32),
            pltpu.VMEM((GRP, CH), jnp.int32),
            pltpu.VMEM((GRP, CH), jnp.int32),
            pltpu.VMEM((CH, D), jnp.float32),
            pltpu.VMEM((CH, D), jnp.float32),
            pltpu.VMEM_SHARED((NPAD, D), jnp.float32),
            pltpu.SemaphoreType.DMA,
            pltpu.SemaphoreType.DMA,
            pltpu.SemaphoreType.DMA,
            pltpu.SemaphoreType.DMA,
        ],
    )(_agg_body)


def _agg_body(hp_hbm, src_hbm, dst_hbm, out_hbm, src_a, dst_a, src_b, dst_b,
              rows_a, rows_b, acc_sh, sem_ga, sem_gb, sem_ia, sem_ib):
    c = lax.axis_index("c")
    s = lax.axis_index("s")
    hp_c = hp_hbm.at[c]

    @pl.when(s < 15)
    def _():
        pltpu.sync_copy(hp_c.at[pl.ds(s * CB, CB)], acc_sh.at[pl.ds(s * CB, CB)])

    @pl.when(s == 15)
    def _():
        pltpu.sync_copy(hp_c.at[pl.ds(15 * CB, CBL)], acc_sh.at[pl.ds(15 * CB, CBL)])

    plsc.subcore_barrier()

    def load_idx(g, sv, dv, sem):
        pltpu.async_copy(src_hbm.at[c, s, pl.ds(g * GRP, GRP)], sv, sem)
        pltpu.async_copy(dst_hbm.at[c, s, pl.ds(g * GRP, GRP)], dv, sem)

    def wait_idx(g, sv, dv, sem):
        pltpu.make_async_copy(src_hbm.at[c, s, pl.ds(g * GRP, GRP)], sv, sem).wait()
        pltpu.make_async_copy(dst_hbm.at[c, s, pl.ds(g * GRP, GRP)], dv, sem).wait()

    def run_group(src_v, dst_v):
        pltpu.async_copy(hp_c.at[src_v.at[0]], rows_a, sem_ga)

        def pair(kk, carry2):
            k0 = kk * 2
            pltpu.async_copy(hp_c.at[src_v.at[k0 + 1]], rows_b, sem_gb)
            pltpu.make_async_copy(hp_c.at[src_v.at[k0]], rows_a, sem_ga).wait()
            pltpu.sync_copy(rows_a, acc_sh.at[dst_v.at[k0]], add=True)

            @pl.when(kk < GRP // 2 - 1)
            def _():
                pltpu.async_copy(hp_c.at[src_v.at[k0 + 2]], rows_a, sem_ga)

            pltpu.make_async_copy(hp_c.at[src_v.at[k0 + 1]], rows_b, sem_gb).wait()
            pltpu.sync_copy(rows_b, acc_sh.at[dst_v.at[k0 + 1]], add=True)
            return carry2

        lax.fori_loop(0, GRP // 2, pair, 0)

    load_idx(0, src_a, dst_a, sem_ia)

    def grp_pair(gg, carry):
        g0 = gg * 2
        wait_idx(g0, src_a, dst_a, sem_ia)
        pltpu.async_copy(src_hbm.at[c, s, pl.ds((g0 + 1) * GRP, GRP)], src_b, sem_ib)
        pltpu.async_copy(dst_hbm.at[c, s, pl.ds((g0 + 1) * GRP, GRP)], dst_b, sem_ib)
        run_group(src_a, dst_a)
        wait_idx(g0 + 1, src_b, dst_b, sem_ib)

        @pl.when(gg < NGRP // 2 - 1)
        def _():
            load_idx(g0 + 2, src_a, dst_a, sem_ia)

        run_group(src_b, dst_b)
        return carry

    lax.fori_loop(0, NGRP // 2, grp_pair, 0)
    plsc.subcore_barrier()

    @pl.when(s < 15)
    def _():
        pltpu.sync_copy(acc_sh.at[pl.ds(s * CB, CB)], out_hbm.at[c, pl.ds(s * CB, CB)])

    @pl.when(s == 15)
    def _():
        pltpu.sync_copy(acc_sh.at[pl.ds(15 * CB, CBL)], out_hbm.at[c, pl.ds(15 * CB, CBL)])


_BM = 2000
_NB = N // _BM


def _tc1_body(x_ref, w_ref, deg_ref, hp_ref):
    dv = lax.rsqrt(deg_ref[0])
    h = jnp.dot(x_ref[0], w_ref[...], preferred_element_type=jnp.float32)
    hp_ref[0] = h * dv


def _tc1(x, w1, deg):
    return pl.pallas_call(
        _tc1_body,
        grid=(2, _NB),
        in_specs=[
            pl.BlockSpec((1, _BM, D), lambda g, i: (g, i, 0)),
            pl.BlockSpec((D, D), lambda g, i: (0, 0)),
            pl.BlockSpec((1, _BM, D), lambda g, i: (g, i, 0)),
        ],
        out_specs=pl.BlockSpec((1, _BM, D), lambda g, i: (g, i, 0)),
        out_shape=jax.ShapeDtypeStruct((2, N, D), jnp.float32),
    )(x, w1, deg)


def _tc2_body(agg_ref, deg_ref, b_ref, w_ref, out_ref):
    dv = lax.rsqrt(deg_ref[0])
    y = jnp.maximum(agg_ref[0] * dv + b_ref[...], 0.0)
    out_ref[0] = jnp.dot(y, w_ref[...], preferred_element_type=jnp.float32) * dv


def _tc2(agg, deg, b1, w2):
    return pl.pallas_call(
        _tc2_body,
        grid=(2, _NB),
        in_specs=[
            pl.BlockSpec((1, _BM, D), lambda g, i: (g, i, 0)),
            pl.BlockSpec((1, _BM, D), lambda g, i: (g, i, 0)),
            pl.BlockSpec((1, D), lambda g, i: (0, 0)),
            pl.BlockSpec((D, D), lambda g, i: (0, 0)),
        ],
        out_specs=pl.BlockSpec((1, _BM, D), lambda g, i: (g, i, 0)),
        out_shape=jax.ShapeDtypeStruct((2, N, D), jnp.float32),
    )(agg, deg, b1, w2)


def _tc3_body(agg_ref, deg_ref, b_ref, out_ref, sum_ref, sq_ref):
    p = pl.program_id(1)
    i = pl.program_id(2)
    dv = lax.rsqrt(deg_ref[0])
    h = agg_ref[0] * dv + b_ref[...]

    @pl.when((p == 0) & (i == 0))
    def _():
        sum_ref[...] = jnp.zeros_like(sum_ref)
        sq_ref[...] = jnp.zeros_like(sq_ref)

    @pl.when(p == 0)
    def _():
        sum_ref[...] += jnp.sum(h, axis=0, keepdims=True)
        sq_ref[...] += jnp.sum(h * h, axis=0, keepdims=True)
        out_ref[0] = h

    @pl.when(p == 1)
    def _():
        mean = sum_ref[...] / N
        var = (sq_ref[...] - N * mean * mean) / (N - 1)
        out_ref[0] = (h - mean) * lax.rsqrt(var)


def _tc3(agg, deg, b2):
    return pl.pallas_call(
        _tc3_body,
        grid=(2, 2, _NB),
        in_specs=[
            pl.BlockSpec((1, _BM, D), lambda g, p, i: (g, i, 0)),
            pl.BlockSpec((1, _BM, D), lambda g, p, i: (g, i, 0)),
            pl.BlockSpec((1, D), lambda g, p, i: (0, 0)),
        ],
        out_specs=pl.BlockSpec((1, _BM, D), lambda g, p, i: (g, i, 0)),
        out_shape=jax.ShapeDtypeStruct((2, N, D), jnp.float32),
        scratch_shapes=[
            pltpu.VMEM((1, D), jnp.float32),
            pltpu.VMEM((1, D), jnp.float32),
        ],
    )(agg, deg, b2)


def kernel(x1, x2, edge_index1, edge_index2, W1, b1, W2, b2):
    src = jnp.stack([edge_index1[0], edge_index2[0]])
    dst = jnp.stack([edge_index1[1], edge_index2[1]])
    extra = EPAD - E
    pad_src = (jnp.arange(extra, dtype=jnp.int32) * 97) % N
    pad_dst = N + (jnp.arange(extra, dtype=jnp.int32) % (NPAD - N))
    srcp = jnp.concatenate(
        [src, jnp.broadcast_to(pad_src, (2, extra))], axis=1
    ).reshape(2, NT, NCH, CH)
    dstp = jnp.concatenate(
        [dst, jnp.broadcast_to(pad_dst, (2, extra))], axis=1
    ).reshape(2, NT, NCH, CH)

    ones = jnp.ones((N, D), jnp.float32)
    x = jnp.stack([x1, x2])

    deg = _get_deg_kernel()(dstp, ones)
    hp1 = _tc1(x, W1, deg)
    agg = _get_agg_kernel()
    agg1 = agg(hp1, srcp, dstp)
    hp2 = _tc2(agg1, deg, b1.reshape(1, D), W2)
    agg2 = agg(hp2, srcp, dstp)
    z = _tc3(agg2, deg, b2.reshape(1, D))
    return z[0], z[1]


# speedup vs baseline: 1.2799x; 1.0012x over previous
import functools

import jax
import jax.numpy as jnp
from jax import lax
from jax.experimental import pallas as pl
from jax.experimental.pallas import tpu as pltpu
from jax.experimental.pallas import tpu_sc as plsc

N = 10000
E = 320000
D = 128
NT = 16
CH = 128
NCH = 160
GRP = 16
NGRP = NCH // GRP
EPT = NCH * CH
EPAD = NT * EPT
NPAD = 10016
CB = 632
CBL = N - 15 * CB


@functools.cache
def _get_deg_kernel():
    mesh = plsc.VectorSubcoreMesh(core_axis_name="c", subcore_axis_name="s")
    return functools.partial(
        pl.kernel,
        mesh=mesh,
        out_type=jax.ShapeDtypeStruct((2, N, D), jnp.float32),
        scratch_types=[
            pltpu.VMEM((GRP, CH), jnp.int32),
            pltpu.VMEM((GRP, CH), jnp.int32),
            pltpu.VMEM((CH, D), jnp.float32),
            pltpu.VMEM_SHARED((NPAD, D), jnp.float32),
            pltpu.SemaphoreType.DMA,
            pltpu.SemaphoreType.DMA,
        ],
    )(_deg_body)


def _deg_body(dst_hbm, ones_hbm, out_hbm, dst_a, dst_b, rows_v, acc_sh, sem_ia, sem_ib):
    c = lax.axis_index("c")
    s = lax.axis_index("s")
    pltpu.sync_copy(ones_hbm.at[pl.ds(0, CH)], rows_v)

    @pl.when(s < 15)
    def _():
        pltpu.sync_copy(ones_hbm.at[pl.ds(s * CB, CB)], acc_sh.at[pl.ds(s * CB, CB)])

    @pl.when(s == 15)
    def _():
        pltpu.sync_copy(ones_hbm.at[pl.ds(15 * CB, CBL)], acc_sh.at[pl.ds(15 * CB, CBL)])

    plsc.subcore_barrier()

    def run_group(dst_v):
        def body(k, carry2):
            pltpu.sync_copy(rows_v, acc_sh.at[dst_v.at[k]], add=True)
            return carry2

        lax.fori_loop(0, GRP, body, 0)

    pltpu.async_copy(dst_hbm.at[c, s, pl.ds(0, GRP)], dst_a, sem_ia)

    def grp_pair(gg, carry):
        g0 = gg * 2
        pltpu.make_async_copy(dst_hbm.at[c, s, pl.ds(g0 * GRP, GRP)], dst_a, sem_ia).wait()
        pltpu.async_copy(dst_hbm.at[c, s, pl.ds((g0 + 1) * GRP, GRP)], dst_b, sem_ib)
        run_group(dst_a)
        pltpu.make_async_copy(dst_hbm.at[c, s, pl.ds((g0 + 1) * GRP, GRP)], dst_b, sem_ib).wait()

        @pl.when(gg < NGRP // 2 - 1)
        def _():
            pltpu.async_copy(dst_hbm.at[c, s, pl.ds((g0 + 2) * GRP, GRP)], dst_a, sem_ia)

        run_group(dst_b)
        return carry

    lax.fori_loop(0, NGRP // 2, grp_pair, 0)
    plsc.subcore_barrier()

    @pl.when(s < 15)
    def _():
        pltpu.sync_copy(acc_sh.at[pl.ds(s * CB, CB)], out_hbm.at[c, pl.ds(s * CB, CB)])

    @pl.when(s == 15)
    def _():
        pltpu.sync_copy(acc_sh.at[pl.ds(15 * CB, CBL)], out_hbm.at[c, pl.ds(15 * CB, CBL)])


@functools.cache
def _get_agg_kernel():
    mesh = plsc.VectorSubcoreMesh(core_axis_name="c", subcore_axis_name="s")
    return functools.partial(
        pl.kernel,
        mesh=mesh,
        out_type=jax.ShapeDtypeStruct((2, N, D), jnp.float32),
        scratch_types=[
            pltpu.VMEM((GRP, CH), jnp.int32),
            pltpu.VMEM((GRP, CH), jnp.int32),
            pltpu.VMEM((GRP, CH), jnp.int32),
            pltpu.VMEM((GRP, CH), jnp.int32),
            pltpu.VMEM((CH, D), jnp.float32),
            pltpu.VMEM((CH, D), jnp.float32),
            pltpu.VMEM_SHARED((NPAD, D), jnp.float32),
            pltpu.SemaphoreType.DMA,
            pltpu.SemaphoreType.DMA,
            pltpu.SemaphoreType.DMA,
            pltpu.SemaphoreType.DMA,
        ],
    )(_agg_body)


def _agg_body(hp_hbm, src_hbm, dst_hbm, out_hbm, src_a, dst_a, src_b, dst_b,
              rows_a, rows_b, acc_sh, sem_ga, sem_gb, sem_ia, sem_ib):
    c = lax.axis_index("c")
    s = lax.axis_index("s")
    hp_c = hp_hbm.at[c]

    @pl.when(s < 15)
    def _():
        pltpu.sync_copy(hp_c.at[pl.ds(s * CB, CB)], acc_sh.at[pl.ds(s * CB, CB)])

    @pl.when(s == 15)
    def _():
        pltpu.sync_copy(hp_c.at[pl.ds(15 * CB, CBL)], acc_sh.at[pl.ds(15 * CB, CBL)])

    plsc.subcore_barrier()

    def load_idx(g, sv, dv, sem):
        pltpu.async_copy(src_hbm.at[c, s, pl.ds(g * GRP, GRP)], sv, sem)
        pltpu.async_copy(dst_hbm.at[c, s, pl.ds(g * GRP, GRP)], dv, sem)

    def wait_idx(g, sv, dv, sem):
        pltpu.make_async_copy(src_hbm.at[c, s, pl.ds(g * GRP, GRP)], sv, sem).wait()
        pltpu.make_async_copy(dst_hbm.at[c, s, pl.ds(g * GRP, GRP)], dv, sem).wait()

    def run_group(src_v, dst_v):
        pltpu.async_copy(hp_c.at[src_v.at[0]], rows_a, sem_ga)

        def pair(kk, carry2):
            k0 = kk * 2
            pltpu.async_copy(hp_c.at[src_v.at[k0 + 1]], rows_b, sem_gb)
            pltpu.make_async_copy(hp_c.at[src_v.at[k0]], rows_a, sem_ga).wait()
            pltpu.sync_copy(rows_a, acc_sh.at[dst_v.at[k0]], add=True)

            @pl.when(kk < GRP // 2 - 1)
            def _():
                pltpu.async_copy(hp_c.at[src_v.at[k0 + 2]], rows_a, sem_ga)

            pltpu.make_async_copy(hp_c.at[src_v.at[k0 + 1]], rows_b, sem_gb).wait()
            pltpu.sync_copy(rows_b, acc_sh.at[dst_v.at[k0 + 1]], add=True)
            return carry2

        lax.fori_loop(0, GRP // 2, pair, 0)

    load_idx(0, src_a, dst_a, sem_ia)

    def grp_pair(gg, carry):
        g0 = gg * 2
        wait_idx(g0, src_a, dst_a, sem_ia)
        pltpu.async_copy(src_hbm.at[c, s, pl.ds((g0 + 1) * GRP, GRP)], src_b, sem_ib)
        pltpu.async_copy(dst_hbm.at[c, s, pl.ds((g0 + 1) * GRP, GRP)], dst_b, sem_ib)
        run_group(src_a, dst_a)
        wait_idx(g0 + 1, src_b, dst_b, sem_ib)

        @pl.when(gg < NGRP // 2 - 1)
        def _():
            load_idx(g0 + 2, src_a, dst_a, sem_ia)

        run_group(src_b, dst_b)
        return carry

    lax.fori_loop(0, NGRP // 2, grp_pair, 0)
    plsc.subcore_barrier()

    @pl.when(s < 15)
    def _():
        pltpu.sync_copy(acc_sh.at[pl.ds(s * CB, CB)], out_hbm.at[c, pl.ds(s * CB, CB)])

    @pl.when(s == 15)
    def _():
        pltpu.sync_copy(acc_sh.at[pl.ds(15 * CB, CBL)], out_hbm.at[c, pl.ds(15 * CB, CBL)])


_BM = 2000
_NB = N // _BM


def _tc1_body(x_ref, w_ref, deg_ref, hp_ref):
    dv = lax.rsqrt(deg_ref[0])
    h = jnp.dot(x_ref[0], w_ref[...], preferred_element_type=jnp.float32)
    hp_ref[0] = h * dv


def _tc1(x, w1, deg):
    return pl.pallas_call(
        _tc1_body,
        grid=(2, _NB),
        in_specs=[
            pl.BlockSpec((1, _BM, D), lambda g, i: (g, i, 0)),
            pl.BlockSpec((D, D), lambda g, i: (0, 0)),
            pl.BlockSpec((1, _BM, D), lambda g, i: (g, i, 0)),
        ],
        out_specs=pl.BlockSpec((1, _BM, D), lambda g, i: (g, i, 0)),
        out_shape=jax.ShapeDtypeStruct((2, N, D), jnp.float32),
    )(x, w1, deg)


def _tc2_body(agg_ref, deg_ref, b_ref, w_ref, out_ref):
    dv = lax.rsqrt(deg_ref[0])
    y = jnp.maximum(agg_ref[0] * dv + b_ref[...], 0.0)
    out_ref[0] = jnp.dot(y, w_ref[...], preferred_element_type=jnp.float32) * dv


def _tc2(agg, deg, b1, w2):
    return pl.pallas_call(
        _tc2_body,
        grid=(2, _NB),
        in_specs=[
            pl.BlockSpec((1, _BM, D), lambda g, i: (g, i, 0)),
            pl.BlockSpec((1, _BM, D), lambda g, i: (g, i, 0)),
            pl.BlockSpec((1, D), lambda g, i: (0, 0)),
            pl.BlockSpec((D, D), lambda g, i: (0, 0)),
        ],
        out_specs=pl.BlockSpec((1, _BM, D), lambda g, i: (g, i, 0)),
        out_shape=jax.ShapeDtypeStruct((2, N, D), jnp.float32),
    )(agg, deg, b1, w2)


def _tc3_body(agg_ref, deg_ref, b_ref, out_ref, sum_ref, sq_ref):
    p = pl.program_id(1)
    i = pl.program_id(2)
    dv = lax.rsqrt(deg_ref[0])
    h = agg_ref[0] * dv + b_ref[...]

    @pl.when((p == 0) & (i == 0))
    def _():
        sum_ref[...] = jnp.zeros_like(sum_ref)
        sq_ref[...] = jnp.zeros_like(sq_ref)

    @pl.when(p == 0)
    def _():
        sum_ref[...] += jnp.sum(h, axis=0, keepdims=True)
        sq_ref[...] += jnp.sum(h * h, axis=0, keepdims=True)
        out_ref[0] = h

    @pl.when(p == 1)
    def _():
        mean = sum_ref[...] / N
        var = (sq_ref[...] - N * mean * mean) / (N - 1)
        out_ref[0] = (h - mean) * lax.rsqrt(var)


def _tc3(agg, deg, b2):
    return pl.pallas_call(
        _tc3_body,
        grid=(2, 2, _NB),
        in_specs=[
            pl.BlockSpec((1, _BM, D), lambda g, p, i: (g, i, 0)),
            pl.BlockSpec((1, _BM, D), lambda g, p, i: (g, i, 0)),
            pl.BlockSpec((1, D), lambda g, p, i: (0, 0)),
        ],
        out_specs=pl.BlockSpec((1, _BM, D), lambda g, p, i: (g, i, 0)),
        out_shape=jax.ShapeDtypeStruct((2, N, D), jnp.float32),
        scratch_shapes=[
            pltpu.VMEM((1, D), jnp.float32),
            pltpu.VMEM((1, D), jnp.float32),
        ],
    )(agg, deg, b2)


def kernel(x1, x2, edge_index1, edge_index2, W1, b1, W2, b2):
    src = jnp.stack([edge_index1[0], edge_index2[0]])
    dst = jnp.stack([edge_index1[1], edge_index2[1]])
    extra = EPAD - E
    pad_src = (jnp.arange(extra, dtype=jnp.int32) * 97) % N
    pad_dst = N + (jnp.arange(extra, dtype=jnp.int32) % (NPAD - N))
    srcp = jnp.concatenate(
        [src, jnp.broadcast_to(pad_src, (2, extra))], axis=1
    ).reshape(2, NT, NCH, CH)
    dstp = jnp.concatenate(
        [dst, jnp.broadcast_to(pad_dst, (2, extra))], axis=1
    ).reshape(2, NT, NCH, CH)

    ones = jnp.ones((N, D), jnp.float32)
    x = jnp.stack([x1, x2])

    deg = _get_deg_kernel()(dstp, ones)
    hp1 = _tc1(x, W1, deg)
    agg = _get_agg_kernel()
    agg1 = agg(hp1, srcp, dstp)
    hp2 = _tc2(agg1, deg, b1.reshape(1, D), W2)
    agg2 = agg(hp2, srcp, dstp)
    z = _tc3(agg2, deg, b2.reshape(1, D))
    return z[0], z[1]
